# Initial kernel scaffold; baseline (speedup 1.0000x reference)
import jax
import jax.numpy as jnp
from jax.experimental import pallas as pl


def kernel(voxcam_coords, img_feats, vox_valid, W_comp):
    raise NotImplementedError("write your pallas kernel here")


# speedup vs baseline: 29.7807x; 29.7807x over previous
import functools

import jax
import jax.numpy as jnp
from jax.experimental import pallas as pl


def _fold_weights_kernel(img_ref, wy_ref, g_ref):
    n = img_ref.shape[0]
    ycam = wy_ref.shape[0]
    for ni in range(n):
        for yi in range(ycam):
            g_ref[ni, yi] = jax.lax.dot(
                wy_ref[yi], img_ref[ni], preferred_element_type=jnp.float32
            ).astype(jnp.bfloat16)


def _sample_kernel(ix_ref, iy_ref, scale_ref, valid_ref, g_ref, out_ref, *, h, w):
    n, ycam, p = ix_ref.shape
    hw = h * w
    out_c = out_ref.shape[0]
    maskf = (valid_ref[...] > 0.0).astype(jnp.float32)
    cnt = jnp.sum(maskf, axis=0)
    invcnt = 1.0 / jnp.maximum(cnt, 1.0)
    jf = jax.lax.broadcasted_iota(jnp.int32, (hw, 1), 0).astype(jnp.float32)
    jh = jnp.floor((jf + 0.5) * (1.0 / w))
    jw = jf - w * jh
    acc = jnp.zeros((out_c, p), jnp.float32)
    for yi in range(ycam):
        acc_y = jnp.zeros((out_c, p), jnp.float32)
        for ni in range(n):
            ixr = ix_ref[ni, yi : yi + 1, :]
            iyr = iy_ref[ni, yi : yi + 1, :]
            scr = scale_ref[ni, yi : yi + 1, :]
            tent_h = jnp.maximum(1.0 - jnp.abs(jh - iyr), 0.0)
            tent_w = jnp.maximum(1.0 - jnp.abs(jw - ixr), 0.0)
            a_t = (tent_h * tent_w * scr).astype(jnp.bfloat16)
            acc_y = acc_y + jax.lax.dot(
                g_ref[ni, yi], a_t, preferred_element_type=jnp.float32
            )
        acc = acc + acc_y * invcnt[yi : yi + 1, :]
    out_ref[...] = acc


def kernel(voxcam_coords, img_feats, vox_valid, W_comp):
    bt, n, zcam, ycam, xcam, _ = voxcam_coords.shape
    c, h, w = img_feats.shape[2:]
    out_c = W_comp.shape[0]
    hw = h * w
    ptot = zcam * xcam
    p = 2560
    pad = (-ptot) % p
    nblk = (ptot + pad) // p

    wy = jnp.transpose(W_comp.reshape(out_c, c, ycam), (2, 0, 1)).astype(jnp.bfloat16)
    outs = []
    for b in range(bt):
        img = img_feats[b].reshape(n, c, hw).astype(jnp.bfloat16)
        g = pl.pallas_call(
            _fold_weights_kernel,
            out_shape=jax.ShapeDtypeStruct((n, ycam, out_c, hw), jnp.bfloat16),
        )(img, wy)
        coords = voxcam_coords[b]
        ix = (coords[..., 0] + 1.0) * (w * 0.5) - 0.5
        iy = (coords[..., 1] + 1.0) * (h * 0.5) - 0.5
        wzs = jnp.maximum(1.0 - 0.5 * jnp.abs(coords[..., 2]), 0.0)
        valid = vox_valid[b, ..., 0]
        scale = wzs * (valid > 0.0)

        def _flat(arr):
            flat = jnp.transpose(arr, (0, 2, 1, 3)).reshape(n, ycam, ptot)
            return jnp.pad(flat, ((0, 0), (0, 0), (0, pad)))

        out_t = pl.pallas_call(
            functools.partial(_sample_kernel, h=h, w=w),
            grid=(nblk,),
            in_specs=[
                pl.BlockSpec((n, ycam, p), lambda i: (0, 0, i)),
                pl.BlockSpec((n, ycam, p), lambda i: (0, 0, i)),
                pl.BlockSpec((n, ycam, p), lambda i: (0, 0, i)),
                pl.BlockSpec((n, ycam, p), lambda i: (0, 0, i)),
                pl.BlockSpec((n, ycam, out_c, hw), lambda i: (0, 0, 0, 0)),
            ],
            out_specs=pl.BlockSpec((out_c, p), lambda i: (0, i)),
            out_shape=jax.ShapeDtypeStruct((out_c, ptot + pad), jnp.float32),
        )(_flat(ix), _flat(iy), _flat(scale), _flat(valid), g)
        outs.append(out_t[:, :ptot].reshape(out_c, zcam, xcam))
    return jnp.stack(outs, axis=0)


# speedup vs baseline: 50.9860x; 1.7121x over previous
import functools

import jax
import jax.numpy as jnp
from jax.experimental import pallas as pl


def _fold_weights_kernel(img_ref, wy_ref, g_ref):
    n = img_ref.shape[0]
    ycam = wy_ref.shape[0]
    for ni in range(n):
        for yi in range(ycam):
            g_ref[ni, yi] = jax.lax.dot(
                wy_ref[yi], img_ref[ni], preferred_element_type=jnp.float32
            ).astype(jnp.bfloat16)


def _sample_kernel(ix_ref, iy_ref, scale_ref, valid_ref, g_ref, ry_ref, out_ref, *, h, w):
    n, ycam, p = ix_ref.shape
    hw = h * w
    out_c = out_ref.shape[0]
    maskf = (valid_ref[...] > 0.0).astype(jnp.float32)
    cnt = jnp.sum(maskf, axis=0)
    invcnt = 1.0 / jnp.maximum(cnt, 1.0)
    hi = jax.lax.broadcasted_iota(jnp.int32, (h, 1), 0).astype(jnp.float32)
    wi = jax.lax.broadcasted_iota(jnp.int32, (w, 1), 0).astype(jnp.float32)
    acc = jnp.zeros((out_c, p), jnp.float32)
    for yi in range(ycam):
        acc_y = jnp.zeros((out_c, p), jnp.float32)
        for ni in range(n):
            ixr = ix_ref[ni, yi : yi + 1, :]
            iyr = iy_ref[ni, yi : yi + 1, :]
            scr = scale_ref[ni, yi : yi + 1, :]
            tent_h = jnp.maximum(1.0 - jnp.abs(hi - iyr), 0.0)
            tent_w = jnp.maximum(1.0 - jnp.abs(wi - ixr), 0.0) * scr
            th_big = jax.lax.dot(
                ry_ref[...], tent_h.astype(jnp.bfloat16),
                preferred_element_type=jnp.float32,
            )
            tw_big = jnp.concatenate([tent_w] * h, axis=0)
            a_t = (th_big * tw_big).astype(jnp.bfloat16)
            acc_y = acc_y + jax.lax.dot(
                g_ref[ni, yi], a_t, preferred_element_type=jnp.float32
            )
        acc = acc + acc_y * invcnt[yi : yi + 1, :]
    out_ref[...] = acc


def kernel(voxcam_coords, img_feats, vox_valid, W_comp):
    bt, n, zcam, ycam, xcam, _ = voxcam_coords.shape
    c, h, w = img_feats.shape[2:]
    out_c = W_comp.shape[0]
    hw = h * w
    ptot = zcam * xcam
    p = 1280
    pad = (-ptot) % p
    nblk = (ptot + pad) // p
    jrow = jnp.arange(hw) // w
    ry = (jrow[:, None] == jnp.arange(h)[None, :]).astype(jnp.bfloat16)

    wy = jnp.transpose(W_comp.reshape(out_c, c, ycam), (2, 0, 1)).astype(jnp.bfloat16)
    outs = []
    for b in range(bt):
        img = img_feats[b].reshape(n, c, hw).astype(jnp.bfloat16)
        g = pl.pallas_call(
            _fold_weights_kernel,
            out_shape=jax.ShapeDtypeStruct((n, ycam, out_c, hw), jnp.bfloat16),
        )(img, wy)
        coords = voxcam_coords[b]
        ix = (coords[..., 0] + 1.0) * (w * 0.5) - 0.5
        iy = (coords[..., 1] + 1.0) * (h * 0.5) - 0.5
        wzs = jnp.maximum(1.0 - 0.5 * jnp.abs(coords[..., 2]), 0.0)
        valid = vox_valid[b, ..., 0]
        scale = wzs * (valid > 0.0)

        def _flat(arr):
            flat = jnp.transpose(arr, (0, 2, 1, 3)).reshape(n, ycam, ptot)
            return jnp.pad(flat, ((0, 0), (0, 0), (0, pad)))

        out_t = pl.pallas_call(
            functools.partial(_sample_kernel, h=h, w=w),
            grid=(nblk,),
            in_specs=[
                pl.BlockSpec((n, ycam, p), lambda i: (0, 0, i)),
                pl.BlockSpec((n, ycam, p), lambda i: (0, 0, i)),
                pl.BlockSpec((n, ycam, p), lambda i: (0, 0, i)),
                pl.BlockSpec((n, ycam, p), lambda i: (0, 0, i)),
                pl.BlockSpec((n, ycam, out_c, hw), lambda i: (0, 0, 0, 0)),
                pl.BlockSpec((hw, h), lambda i: (0, 0)),
            ],
            out_specs=pl.BlockSpec((out_c, p), lambda i: (0, i)),
            out_shape=jax.ShapeDtypeStruct((out_c, ptot + pad), jnp.float32),
        )(_flat(ix), _flat(iy), _flat(scale), _flat(valid), g, ry)
        outs.append(out_t[:, :ptot].reshape(out_c, zcam, xcam))
    return jnp.stack(outs, axis=0)


# speedup vs baseline: 52.2124x; 1.0241x over previous
import functools

import jax
import jax.numpy as jnp
from jax.experimental import pallas as pl


def _fold_weights_kernel(img_ref, wy_ref, g_ref):
    n = img_ref.shape[0]
    ycam = wy_ref.shape[0]
    for ni in range(n):
        for yi in range(ycam):
            g_ref[ni, yi] = jax.lax.dot(
                wy_ref[yi], img_ref[ni], preferred_element_type=jnp.float32
            ).astype(jnp.bfloat16)


def _sample_kernel(ix_ref, iy_ref, scale_ref, valid_ref, g_ref, ry_ref, out_ref, *, h, w):
    n, ycam, p = ix_ref.shape
    hw = h * w
    out_c = out_ref.shape[0]
    maskf = (valid_ref[...] > 0.0).astype(jnp.float32)
    cnt = jnp.sum(maskf, axis=0)
    invcnt = 1.0 / jnp.maximum(cnt, 1.0)
    hi = jax.lax.broadcasted_iota(jnp.int32, (h, 1), 0).astype(jnp.float32)
    wi = jax.lax.broadcasted_iota(jnp.int32, (w, 1), 0).astype(jnp.float32)
    acc = jnp.zeros((out_c, p), jnp.float32)
    for yi in range(ycam):
        acc_y = jnp.zeros((out_c, p), jnp.float32)
        for ni in range(n):
            ixr = ix_ref[ni, yi : yi + 1, :]
            iyr = iy_ref[ni, yi : yi + 1, :]
            scr = scale_ref[ni, yi : yi + 1, :]
            tent_h = jnp.maximum(1.0 - jnp.abs(hi - iyr), 0.0)
            tent_w = jnp.maximum(1.0 - jnp.abs(wi - ixr), 0.0) * scr
            th_big = jax.lax.dot(
                ry_ref[...], tent_h.astype(jnp.bfloat16),
                preferred_element_type=jnp.float32,
            ).astype(jnp.bfloat16)
            tw_big = jnp.concatenate([tent_w.astype(jnp.bfloat16)] * h, axis=0)
            a_t = th_big * tw_big
            acc_y = acc_y + jax.lax.dot(
                g_ref[ni, yi], a_t, preferred_element_type=jnp.float32
            )
        acc = acc + acc_y * invcnt[yi : yi + 1, :]
    out_ref[...] = acc


def kernel(voxcam_coords, img_feats, vox_valid, W_comp):
    bt, n, zcam, ycam, xcam, _ = voxcam_coords.shape
    c, h, w = img_feats.shape[2:]
    out_c = W_comp.shape[0]
    hw = h * w
    ptot = zcam * xcam
    p = 2560
    pad = (-ptot) % p
    nblk = (ptot + pad) // p
    jrow = jnp.arange(hw) // w
    ry = (jrow[:, None] == jnp.arange(h)[None, :]).astype(jnp.bfloat16)

    wy = jnp.transpose(W_comp.reshape(out_c, c, ycam), (2, 0, 1)).astype(jnp.bfloat16)
    outs = []
    for b in range(bt):
        img = img_feats[b].reshape(n, c, hw).astype(jnp.bfloat16)
        g = pl.pallas_call(
            _fold_weights_kernel,
            out_shape=jax.ShapeDtypeStruct((n, ycam, out_c, hw), jnp.bfloat16),
        )(img, wy)
        coords = voxcam_coords[b]
        ix = (coords[..., 0] + 1.0) * (w * 0.5) - 0.5
        iy = (coords[..., 1] + 1.0) * (h * 0.5) - 0.5
        wzs = jnp.maximum(1.0 - 0.5 * jnp.abs(coords[..., 2]), 0.0)
        valid = vox_valid[b, ..., 0]
        scale = wzs * (valid > 0.0)

        def _flat(arr):
            flat = jnp.transpose(arr, (0, 2, 1, 3)).reshape(n, ycam, ptot)
            return jnp.pad(flat, ((0, 0), (0, 0), (0, pad)))

        out_t = pl.pallas_call(
            functools.partial(_sample_kernel, h=h, w=w),
            grid=(nblk,),
            in_specs=[
                pl.BlockSpec((n, ycam, p), lambda i: (0, 0, i)),
                pl.BlockSpec((n, ycam, p), lambda i: (0, 0, i)),
                pl.BlockSpec((n, ycam, p), lambda i: (0, 0, i)),
                pl.BlockSpec((n, ycam, p), lambda i: (0, 0, i)),
                pl.BlockSpec((n, ycam, out_c, hw), lambda i: (0, 0, 0, 0)),
                pl.BlockSpec((hw, h), lambda i: (0, 0)),
            ],
            out_specs=pl.BlockSpec((out_c, p), lambda i: (0, i)),
            out_shape=jax.ShapeDtypeStruct((out_c, ptot + pad), jnp.float32),
        )(_flat(ix), _flat(iy), _flat(scale), _flat(valid), g, ry)
        outs.append(out_t[:, :ptot].reshape(out_c, zcam, xcam))
    return jnp.stack(outs, axis=0)
